# baseline (speedup 1.0000x reference)
import functools

import jax
import jax.numpy as jnp
from jax import lax
from jax.experimental import pallas as pl
from jax.experimental.pallas import tpu as pltpu
from jax.experimental.pallas import tpu_sc as plsc

EPS_LN = 1e-12

_NC = 2
_NS = 16
_NW = _NC * _NS
_GW = 16
_RING = 2
_TB = 512
_CHUNKS = (512, 512, 512, 256, 256)


def _sc_gather(word_emb, flat_ids):
    n = flat_ids.shape[0]
    _, d = word_emb.shape
    b_per_w = n // _NW
    nsub = b_per_w // _GW
    mesh = plsc.VectorSubcoreMesh(core_axis_name="c", subcore_axis_name="s")

    ring = min(_RING, nsub)

    @functools.partial(
        pl.kernel,
        mesh=mesh,
        out_type=jax.ShapeDtypeStruct((n, d), word_emb.dtype),
        scratch_types=[
            pltpu.VMEM((b_per_w,), jnp.int32),
        ]
        + [pltpu.VMEM((_GW, d), word_emb.dtype) for _ in range(ring)]
        + [pltpu.SemaphoreType.DMA for _ in range(2 * ring)],
    )
    def gather_kernel(table_hbm, idx_hbm, out_hbm, idx_v, *scr):
        bufs = scr[:ring]
        gsems = scr[ring:2 * ring]
        osems = scr[2 * ring:]
        wid = lax.axis_index("s") * _NC + lax.axis_index("c")
        base = wid * b_per_w
        pltpu.sync_copy(idx_hbm.at[pl.ds(base, b_per_w)], idx_v)

        def gath(j):
            return pltpu.make_async_copy(
                table_hbm.at[idx_v.at[pl.ds(j * _GW, _GW)]],
                bufs[j % ring],
                gsems[j % ring],
            )

        def wr(j):
            return pltpu.make_async_copy(
                bufs[j % ring],
                out_hbm.at[pl.ds(base + j * _GW, _GW)],
                osems[j % ring],
            )

        for r in range(ring):
            gath(r).start()
        for j in range(nsub):
            gath(j).wait()
            wr(j).start()
            k = j - (ring - 1)
            if k >= 0 and k + ring < nsub:
                wr(k).wait()
                gath(k + ring).start()
        for j in range(max(0, nsub - ring), nsub):
            wr(j).wait()

    return gather_kernel(word_emb, flat_ids)


def _ln_body_first(g_ref, pos_ref, tt_ref, tok_ref, gam_ref, bet_ref, o_ref):
    e = g_ref[...] + pos_ref[...]
    w = tt_ref[...].astype(jnp.float32)
    e = e + (tok_ref[0:1, :] + w * (tok_ref[1:2, :] - tok_ref[0:1, :]))
    mu = jnp.mean(e, axis=1, keepdims=True)
    m2 = jnp.mean(e * e, axis=1, keepdims=True)
    a = lax.rsqrt(m2 - mu * mu + EPS_LN)
    o_ref[...] = (e - mu) * a * gam_ref[...] + bet_ref[...]


def _ln_body_next(prev_ref, g_ref, pos_ref, tt_ref, tok_ref, gam_ref, bet_ref,
                  o_ref):
    del prev_ref
    _ln_body_first(g_ref, pos_ref, tt_ref, tok_ref, gam_ref, bet_ref, o_ref)


def _tc_ln_chunk(out_prev, gathered, tt_w, pos_emb, tok_emb, gamma2d, beta2d,
                 s_start, n_s_total, batch, total_n):
    n_ck, h = gathered.shape
    sw = n_ck // batch
    tb = min(_TB, sw)
    n_sc = sw // tb
    blk0 = s_start // tb
    n_blk_total = (n_s_total * _TB) // tb
    grid = (n_sc, batch)
    data_specs = [
        pl.BlockSpec((tb, h), lambda i, bb: (bb * n_sc + i, 0)),
        pl.BlockSpec((tb, h), lambda i, bb: (blk0 + i, 0)),
        pl.BlockSpec((tb, 1), lambda i, bb: (bb * n_sc + i, 0)),
        pl.BlockSpec(tok_emb.shape, lambda i, bb: (0, 0)),
        pl.BlockSpec((1, h), lambda i, bb: (0, 0)),
        pl.BlockSpec((1, h), lambda i, bb: (0, 0)),
    ]
    out_spec = pl.BlockSpec(
        (tb, h), lambda i, bb: (bb * n_blk_total + blk0 + i, 0)
    )
    out_shape = jax.ShapeDtypeStruct((total_n, h), jnp.float32)
    data = (gathered, pos_emb, tt_w, tok_emb, gamma2d, beta2d)
    if out_prev is None:
        return pl.pallas_call(
            _ln_body_first,
            grid=grid,
            in_specs=data_specs,
            out_specs=out_spec,
            out_shape=out_shape,
        )(*data)
    return pl.pallas_call(
        _ln_body_next,
        grid=grid,
        in_specs=[pl.BlockSpec(memory_space=pl.ANY)] + data_specs,
        out_specs=out_spec,
        out_shape=out_shape,
        input_output_aliases={0: 0},
    )(out_prev, *data)


def kernel(input_ids, token_type_ids, word_emb, pos_emb, tok_emb, gamma, beta):
    b, s = input_ids.shape
    h = word_emb.shape[1]
    ids = input_ids.astype(jnp.int32)
    tt_i = token_type_ids.astype(jnp.int32)
    pos = pos_emb[:s]
    gamma2d = gamma.reshape(1, -1)
    beta2d = beta.reshape(1, -1)
    n_s_total = s // _TB
    out = None
    s_start = 0
    for sw in _CHUNKS:
        ids_c = ids[:, s_start:s_start + sw].reshape(-1)
        tt_c = tt_i[:, s_start:s_start + sw].reshape(-1, 1)
        gathered = _sc_gather(word_emb, ids_c)
        out = _tc_ln_chunk(out, gathered, tt_c, pos, tok_emb, gamma2d, beta2d,
                           s_start, n_s_total, b, b * s)
        s_start += sw
    return out.reshape(b, s, h)


# speedup vs baseline: 1.0029x; 1.0029x over previous
import functools

import jax
import jax.numpy as jnp
from jax import lax
from jax.experimental import pallas as pl
from jax.experimental.pallas import tpu as pltpu
from jax.experimental.pallas import tpu_sc as plsc

EPS_LN = 1e-12

_NC = 2
_NS = 16
_NW = _NC * _NS
_GW = 16
_RING = 2
_TB = 512
_CHUNKS = (512, 512, 512, 512)


def _sc_gather(word_emb, flat_ids):
    n = flat_ids.shape[0]
    _, d = word_emb.shape
    b_per_w = n // _NW
    nsub = b_per_w // _GW
    mesh = plsc.VectorSubcoreMesh(core_axis_name="c", subcore_axis_name="s")

    ring = min(_RING, nsub)

    @functools.partial(
        pl.kernel,
        mesh=mesh,
        out_type=jax.ShapeDtypeStruct((n, d), word_emb.dtype),
        scratch_types=[
            pltpu.VMEM((b_per_w,), jnp.int32),
        ]
        + [pltpu.VMEM((_GW, d), word_emb.dtype) for _ in range(ring)]
        + [pltpu.SemaphoreType.DMA for _ in range(2 * ring)],
    )
    def gather_kernel(table_hbm, idx_hbm, out_hbm, idx_v, *scr):
        bufs = scr[:ring]
        gsems = scr[ring:2 * ring]
        osems = scr[2 * ring:]
        wid = lax.axis_index("s") * _NC + lax.axis_index("c")
        base = wid * b_per_w
        pltpu.sync_copy(idx_hbm.at[pl.ds(base, b_per_w)], idx_v)

        def gath(j):
            return pltpu.make_async_copy(
                table_hbm.at[idx_v.at[pl.ds(j * _GW, _GW)]],
                bufs[j % ring],
                gsems[j % ring],
            )

        def wr(j):
            return pltpu.make_async_copy(
                bufs[j % ring],
                out_hbm.at[pl.ds(base + j * _GW, _GW)],
                osems[j % ring],
            )

        for r in range(ring):
            gath(r).start()
        for j in range(nsub):
            gath(j).wait()
            wr(j).start()
            k = j - (ring - 1)
            if k >= 0 and k + ring < nsub:
                wr(k).wait()
                gath(k + ring).start()
        for j in range(max(0, nsub - ring), nsub):
            wr(j).wait()

    return gather_kernel(word_emb, flat_ids)


def _ln_body_first(g_ref, pos_ref, tt_ref, tok_ref, gam_ref, bet_ref, o_ref):
    e = g_ref[...] + pos_ref[...]
    w = tt_ref[...].astype(jnp.float32)
    e = e + (tok_ref[0:1, :] + w * (tok_ref[1:2, :] - tok_ref[0:1, :]))
    mu = jnp.mean(e, axis=1, keepdims=True)
    m2 = jnp.mean(e * e, axis=1, keepdims=True)
    a = lax.rsqrt(m2 - mu * mu + EPS_LN)
    o_ref[...] = (e - mu) * a * gam_ref[...] + bet_ref[...]


def _ln_body_next(prev_ref, g_ref, pos_ref, tt_ref, tok_ref, gam_ref, bet_ref,
                  o_ref):
    del prev_ref
    _ln_body_first(g_ref, pos_ref, tt_ref, tok_ref, gam_ref, bet_ref, o_ref)


def _tc_ln_chunk(out_prev, gathered, tt_w, pos_emb, tok_emb, gamma2d, beta2d,
                 s_start, n_s_total, batch, total_n):
    n_ck, h = gathered.shape
    sw = n_ck // batch
    tb = min(_TB, sw)
    n_sc = sw // tb
    blk0 = s_start // tb
    n_blk_total = (n_s_total * _TB) // tb
    grid = (n_sc, batch)
    data_specs = [
        pl.BlockSpec((tb, h), lambda i, bb: (bb * n_sc + i, 0)),
        pl.BlockSpec((tb, h), lambda i, bb: (blk0 + i, 0)),
        pl.BlockSpec((tb, 1), lambda i, bb: (bb * n_sc + i, 0)),
        pl.BlockSpec(tok_emb.shape, lambda i, bb: (0, 0)),
        pl.BlockSpec((1, h), lambda i, bb: (0, 0)),
        pl.BlockSpec((1, h), lambda i, bb: (0, 0)),
    ]
    out_spec = pl.BlockSpec(
        (tb, h), lambda i, bb: (bb * n_blk_total + blk0 + i, 0)
    )
    out_shape = jax.ShapeDtypeStruct((total_n, h), jnp.float32)
    data = (gathered, pos_emb, tt_w, tok_emb, gamma2d, beta2d)
    if out_prev is None:
        return pl.pallas_call(
            _ln_body_first,
            grid=grid,
            in_specs=data_specs,
            out_specs=out_spec,
            out_shape=out_shape,
        )(*data)
    return pl.pallas_call(
        _ln_body_next,
        grid=grid,
        in_specs=[pl.BlockSpec(memory_space=pl.ANY)] + data_specs,
        out_specs=out_spec,
        out_shape=out_shape,
        input_output_aliases={0: 0},
    )(out_prev, *data)


def kernel(input_ids, token_type_ids, word_emb, pos_emb, tok_emb, gamma, beta):
    b, s = input_ids.shape
    h = word_emb.shape[1]
    ids = input_ids.astype(jnp.int32)
    tt_i = token_type_ids.astype(jnp.int32)
    pos = pos_emb[:s]
    gamma2d = gamma.reshape(1, -1)
    beta2d = beta.reshape(1, -1)
    n_s_total = s // _TB
    out = None
    s_start = 0
    for sw in _CHUNKS:
        ids_c = ids[:, s_start:s_start + sw].reshape(-1)
        tt_c = tt_i[:, s_start:s_start + sw].reshape(-1, 1)
        gathered = _sc_gather(word_emb, ids_c)
        out = _tc_ln_chunk(out, gathered, tt_c, pos, tok_emb, gamma2d, beta2d,
                           s_start, n_s_total, b, b * s)
        s_start += sw
    return out.reshape(b, s, h)


# speedup vs baseline: 1.0030x; 1.0001x over previous
import functools

import jax
import jax.numpy as jnp
from jax import lax
from jax.experimental import pallas as pl
from jax.experimental.pallas import tpu as pltpu
from jax.experimental.pallas import tpu_sc as plsc

EPS_LN = 1e-12

_NC = 2
_NS = 16
_NW = _NC * _NS
_GW = 16
_RING = 2
_TB = 512
_NCH = 4


def _sc_gather(word_emb, flat_ids):
    n = flat_ids.shape[0]
    _, d = word_emb.shape
    b_per_w = n // _NW
    nsub = b_per_w // _GW
    mesh = plsc.VectorSubcoreMesh(core_axis_name="c", subcore_axis_name="s")

    ring = min(_RING, nsub)

    @functools.partial(
        pl.kernel,
        mesh=mesh,
        out_type=jax.ShapeDtypeStruct((n, d), word_emb.dtype),
        scratch_types=[
            pltpu.VMEM((b_per_w,), jnp.int32),
        ]
        + [pltpu.VMEM((_GW, d), word_emb.dtype) for _ in range(ring)]
        + [pltpu.SemaphoreType.DMA for _ in range(2 * ring)],
    )
    def gather_kernel(table_hbm, idx_hbm, out_hbm, idx_v, *scr):
        bufs = scr[:ring]
        gsems = scr[ring:2 * ring]
        osems = scr[2 * ring:]
        wid = lax.axis_index("s") * _NC + lax.axis_index("c")
        base = wid * b_per_w
        pltpu.sync_copy(idx_hbm.at[pl.ds(base, b_per_w)], idx_v)

        def gath(j):
            return pltpu.make_async_copy(
                table_hbm.at[idx_v.at[pl.ds(j * _GW, _GW)]],
                bufs[j % ring],
                gsems[j % ring],
            )

        def wr(j):
            return pltpu.make_async_copy(
                bufs[j % ring],
                out_hbm.at[pl.ds(base + j * _GW, _GW)],
                osems[j % ring],
            )

        for r in range(ring):
            gath(r).start()
        for j in range(nsub):
            gath(j).wait()
            wr(j).start()
            k = j - (ring - 1)
            if k >= 0 and k + ring < nsub:
                wr(k).wait()
                gath(k + ring).start()
        for j in range(max(0, nsub - ring), nsub):
            wr(j).wait()

    return gather_kernel(word_emb, flat_ids)


def _ln_body_first(g_ref, pos_ref, tt_ref, tok_ref, gam_ref, bet_ref, o_ref):
    e = g_ref[...] + pos_ref[...]
    w = tt_ref[...].astype(jnp.float32)
    e = e + (tok_ref[0:1, :] + w * (tok_ref[1:2, :] - tok_ref[0:1, :]))
    mu = jnp.mean(e, axis=1, keepdims=True)
    m2 = jnp.mean(e * e, axis=1, keepdims=True)
    a = lax.rsqrt(m2 - mu * mu + EPS_LN)
    o_ref[...] = (e - mu) * a * gam_ref[...] + bet_ref[...]


def _ln_body_next(prev_ref, g_ref, pos_ref, tt_ref, tok_ref, gam_ref, bet_ref,
                  o_ref):
    del prev_ref
    _ln_body_first(g_ref, pos_ref, tt_ref, tok_ref, gam_ref, bet_ref, o_ref)


def _tc_ln_chunk(out_prev, gathered, tt_w, pos_emb, tok_emb, gamma2d, beta2d,
                 s_start, n_s_total, batch, total_n):
    n_ck, h = gathered.shape
    sw = n_ck // batch
    tb = min(_TB, sw)
    n_sc = sw // tb
    blk0 = s_start // tb
    n_blk_total = (n_s_total * _TB) // tb
    grid = (n_sc, batch)
    data_specs = [
        pl.BlockSpec((tb, h), lambda i, bb: (bb * n_sc + i, 0)),
        pl.BlockSpec((tb, h), lambda i, bb: (blk0 + i, 0)),
        pl.BlockSpec((tb, 1), lambda i, bb: (bb * n_sc + i, 0)),
        pl.BlockSpec(tok_emb.shape, lambda i, bb: (0, 0)),
        pl.BlockSpec((1, h), lambda i, bb: (0, 0)),
        pl.BlockSpec((1, h), lambda i, bb: (0, 0)),
    ]
    out_spec = pl.BlockSpec(
        (tb, h), lambda i, bb: (bb * n_blk_total + blk0 + i, 0)
    )
    out_shape = jax.ShapeDtypeStruct((total_n, h), jnp.float32)
    data = (gathered, pos_emb, tt_w, tok_emb, gamma2d, beta2d)
    if out_prev is None:
        return pl.pallas_call(
            _ln_body_first,
            grid=grid,
            in_specs=data_specs,
            out_specs=out_spec,
            out_shape=out_shape,
        )(*data)
    return pl.pallas_call(
        _ln_body_next,
        grid=grid,
        in_specs=[pl.BlockSpec(memory_space=pl.ANY)] + data_specs,
        out_specs=out_spec,
        out_shape=out_shape,
        input_output_aliases={0: 0},
    )(out_prev, *data)


def kernel(input_ids, token_type_ids, word_emb, pos_emb, tok_emb, gamma, beta):
    b, s = input_ids.shape
    h = word_emb.shape[1]
    ids = input_ids.astype(jnp.int32)
    tt_i = token_type_ids.astype(jnp.int32)
    pos = pos_emb[:s]
    gamma2d = gamma.reshape(1, -1)
    beta2d = beta.reshape(1, -1)
    n_s_total = s // _TB
    out = None
    s_start = 0
    for sw in (s // _NCH,) * _NCH:
        ids_c = ids[:, s_start:s_start + sw].reshape(-1)
        tt_c = tt_i[:, s_start:s_start + sw].reshape(-1, 1)
        gathered = _sc_gather(word_emb, ids_c)
        out = _tc_ln_chunk(out, gathered, tt_c, pos, tok_emb, gamma2d, beta2d,
                           s_start, n_s_total, b, b * s)
        s_start += sw
    return out.reshape(b, s, h)
